# baseline (speedup 1.0000x reference)
import jax
import jax.numpy as jnp
from jax import lax
from jax.experimental import pallas as pl
from jax.experimental.pallas import tpu as pltpu
from jax.experimental.pallas import tpu_sc as plsc

B = 16384
K = 128
L = 16
NC = 2
NS = 16
NW = NC * NS
RPW = B // NW
CHUNK = 64
NCHUNKS = RPW // CHUNK
CW = CHUNK * K
GROUPS = CHUNK // L


def _sc_body(in_hbm, out_hbm, u0, u1, i0, i1, ob, su0, su1, si0, si1):
    wid = lax.axis_index("s") * NC + lax.axis_index("c")
    base_row = wid * RPW
    ubufs = (u0, u1)
    ibufs = (i0, i1)
    usems = (su0, su1)
    isems = (si0, si1)
    lane = lax.iota(jnp.int32, L)

    def start(c, b):
        off_u = base_row * K + c * CW
        off_i = B * K + off_u
        pltpu.async_copy(in_hbm.at[pl.ds(off_u, CW)], ubufs[b], usems[b])
        pltpu.async_copy(in_hbm.at[pl.ds(off_i, CW)], ibufs[b], isems[b])

    start(0, 0)
    start(1, 1)

    @pl.loop(0, NCHUNKS, step=2)
    def _chunks(g):
        for b in range(2):
            c = g + b
            ub = ubufs[b]
            ib = ibufs[b]
            pltpu.make_async_copy(in_hbm.at[pl.ds(0, CW)], ub, usems[b]).wait()
            pltpu.make_async_copy(in_hbm.at[pl.ds(0, CW)], ib, isems[b]).wait()
            out_base = c * CHUNK

            @plsc.parallel_loop(0, GROUPS)
            def _group(grp):
                gb = grp * (L * K)
                res = jnp.zeros((L,), jnp.float32)
                for r in range(L):
                    ro = gb + r * K
                    acc = ub[pl.ds(ro, L)] * ib[pl.ds(ro, L)]
                    for k in range(1, K // L):
                        acc = acc + ub[pl.ds(ro + L * k, L)] * ib[pl.ds(ro + L * k, L)]
                    res = jnp.where(lane == r, jnp.sum(acc), res)
                ob[pl.ds(out_base + grp * L, L)] = res

            nxt = c + 2

            @pl.when(nxt < NCHUNKS)
            def _prefetch():
                start(nxt, b)

    pltpu.sync_copy(ob, out_hbm.at[pl.ds(base_row, RPW)])


def kernel(inputs):
    flat = jnp.reshape(inputs, (2 * B * K,))
    f = pl.kernel(
        _sc_body,
        out_type=jax.ShapeDtypeStruct((B,), jnp.float32),
        mesh=plsc.VectorSubcoreMesh(
            core_axis_name="c", subcore_axis_name="s",
            num_cores=NC, num_subcores=NS,
        ),
        scratch_types=[
            pltpu.VMEM((CW,), jnp.float32),
            pltpu.VMEM((CW,), jnp.float32),
            pltpu.VMEM((CW,), jnp.float32),
            pltpu.VMEM((CW,), jnp.float32),
            pltpu.VMEM((RPW,), jnp.float32),
            pltpu.SemaphoreType.DMA,
            pltpu.SemaphoreType.DMA,
            pltpu.SemaphoreType.DMA,
            pltpu.SemaphoreType.DMA,
        ],
        compiler_params=pltpu.CompilerParams(needs_layout_passes=False),
    )
    return f(flat)


# speedup vs baseline: 1.2282x; 1.2282x over previous
import jax
import jax.numpy as jnp
from jax import lax
from jax.experimental import pallas as pl
from jax.experimental.pallas import tpu as pltpu
from jax.experimental.pallas import tpu_sc as plsc

B = 16384
K = 128
L = 16
NC = 2
NS = 16
NW = NC * NS
RPW = B // NW
CHUNK = 64
NCHUNKS = RPW // CHUNK
GROUPS = CHUNK // L
TSTRIDE = L + 1


def _sc_body(in_hbm, out_hbm, u0, u1, i0, i1, ob, tbuf, su0, su1, si0, si1):
    wid = lax.axis_index("s") * NC + lax.axis_index("c")
    base_row = wid * RPW
    ubufs = (u0, u1)
    ibufs = (i0, i1)
    usems = (su0, su1)
    isems = (si0, si1)
    lane = lax.iota(jnp.int32, L)
    lane_t = lane * TSTRIDE

    def start(c, b):
        r0 = base_row + c * CHUNK
        pltpu.async_copy(in_hbm.at[0, pl.ds(r0, CHUNK)], ubufs[b], usems[b])
        pltpu.async_copy(in_hbm.at[1, pl.ds(r0, CHUNK)], ibufs[b], isems[b])

    start(0, 0)
    start(1, 1)

    @pl.loop(0, NCHUNKS, step=2)
    def _chunks(g):
        for b in range(2):
            c = g + b
            ub = ubufs[b]
            ib = ibufs[b]
            pltpu.make_async_copy(in_hbm.at[0, pl.ds(0, CHUNK)], ub, usems[b]).wait()
            pltpu.make_async_copy(in_hbm.at[1, pl.ds(0, CHUNK)], ib, isems[b]).wait()
            out_base = c * CHUNK

            @plsc.parallel_loop(0, GROUPS)
            def _group(grp):
                tb = grp * (L * TSTRIDE)
                for r in range(L):
                    rr = grp * L + r
                    acc = ub[rr, pl.ds(0, L)] * ib[rr, pl.ds(0, L)]
                    for k in range(1, K // L):
                        acc = acc + ub[rr, pl.ds(L * k, L)] * ib[rr, pl.ds(L * k, L)]
                    plsc.store_scatter(tbuf, [lane_t + (tb + r)], acc)
                ws = [plsc.load_gather(tbuf, [lane + (tb + TSTRIDE * l)])
                      for l in range(L)]
                while len(ws) > 1:
                    ws = [ws[2 * i] + ws[2 * i + 1] for i in range(len(ws) // 2)]
                ob[pl.ds(out_base + grp * L, L)] = ws[0]

            nxt = c + 2

            @pl.when(nxt < NCHUNKS)
            def _prefetch():
                start(nxt, b)

    pltpu.sync_copy(ob, out_hbm.at[pl.ds(base_row, RPW)])


def kernel(inputs):
    f = pl.kernel(
        _sc_body,
        out_type=jax.ShapeDtypeStruct((B,), jnp.float32),
        mesh=plsc.VectorSubcoreMesh(
            core_axis_name="c", subcore_axis_name="s",
            num_cores=NC, num_subcores=NS,
        ),
        scratch_types=[
            pltpu.VMEM((CHUNK, K), jnp.float32),
            pltpu.VMEM((CHUNK, K), jnp.float32),
            pltpu.VMEM((CHUNK, K), jnp.float32),
            pltpu.VMEM((CHUNK, K), jnp.float32),
            pltpu.VMEM((RPW,), jnp.float32),
            pltpu.VMEM((GROUPS * L * TSTRIDE,), jnp.float32),
            pltpu.SemaphoreType.DMA,
            pltpu.SemaphoreType.DMA,
            pltpu.SemaphoreType.DMA,
            pltpu.SemaphoreType.DMA,
        ],
        compiler_params=pltpu.CompilerParams(needs_layout_passes=False),
    )
    return f(inputs)
